# baseline (device time: 63472 ns/iter reference)
import jax
import jax.numpy as jnp
from jax import lax
from jax.experimental import pallas as pl
from jax.experimental.pallas import tpu as pltpu

N_DEV = 8


def kernel(x, router_W, route_idx, expert_W, shared_W):
    n_tok, d = x.shape
    e_per, _, h_dim = expert_W.shape
    n_exp = N_DEV * e_per

    def body(x_ref, rw_ref, idx_ref, ew_ref, sw_ref, out_ref,
             comm_ref, send_sems, recv_sems):
        my = lax.axis_index("i")
        left = lax.rem(my - 1 + N_DEV, N_DEV)
        right = lax.rem(my + 1, N_DEV)
        partner = lax.rem(my + 4, N_DEV)

        barrier_sem = pltpu.get_barrier_semaphore()
        for nbr in (left, right, partner):
            pl.semaphore_signal(
                barrier_sem, inc=1,
                device_id=(nbr,), device_id_type=pl.DeviceIdType.MESH,
            )
        pl.semaphore_wait(barrier_sem, 3)

        comm_ref[0, :, :, :] = ew_ref[...].astype(jnp.bfloat16)

        cw = [
            pltpu.make_async_remote_copy(
                src_ref=comm_ref.at[h - 1],
                dst_ref=comm_ref.at[h],
                send_sem=send_sems.at[h - 1],
                recv_sem=recv_sems.at[h - 1],
                device_id=(right,),
                device_id_type=pl.DeviceIdType.MESH,
            )
            for h in range(1, 4)
        ]
        ccw = [
            pltpu.make_async_remote_copy(
                src_ref=comm_ref.at[0 if h == 1 else 4 + h - 1],
                dst_ref=comm_ref.at[4 + h],
                send_sem=send_sems.at[4 + h - 1],
                recv_sem=recv_sems.at[4 + h - 1],
                device_id=(left,),
                device_id_type=pl.DeviceIdType.MESH,
            )
            for h in range(1, 4)
        ]
        zx = pltpu.make_async_remote_copy(
            src_ref=comm_ref.at[0],
            dst_ref=comm_ref.at[4],
            send_sem=send_sems.at[3],
            recv_sem=recv_sems.at[3],
            device_id=(partner,),
            device_id_type=pl.DeviceIdType.MESH,
        )

        cw[0].start()
        ccw[0].start()
        zx.start()

        xf = x_ref[...]
        xbf = xf.astype(jnp.bfloat16)

        scores = jnp.dot(xf, rw_ref[...], preferred_element_type=jnp.float32)
        m = jnp.max(scores, axis=-1, keepdims=True)
        p = jnp.exp(scores - m)
        p = p / jnp.sum(p, axis=-1, keepdims=True)

        idx = idx_ref[...]
        col = lax.broadcasted_iota(jnp.int32, (n_tok, n_exp), 1)
        p_top = jnp.sum(jnp.where(col == idx, p, 0.0), axis=-1, keepdims=True)

        acc = jnp.dot(xbf, sw_ref[...].astype(jnp.bfloat16),
                      preferred_element_type=jnp.float32)

        xs_bf = (p_top * xf).astype(jnp.bfloat16)
        zero_bf = jnp.zeros_like(xs_bf)

        def compute_slot(slot, src_dev, acc):
            for j in range(e_per):
                e = src_dev * e_per + j
                xe = jnp.where(idx == e, xs_bf, zero_bf)
                acc = acc + jnp.dot(xe, comm_ref[slot, j],
                                    preferred_element_type=jnp.float32)
            return acc

        acc = compute_slot(0, my, acc)

        for step in range(1, 4):
            cw[step - 1].wait_recv()
            if step < 3:
                cw[step].start()
            ccw[step - 1].wait_recv()
            if step < 3:
                ccw[step].start()
            acc = compute_slot(step, lax.rem(my - step + N_DEV, N_DEV), acc)
            acc = compute_slot(4 + step, lax.rem(my + step, N_DEV), acc)
            if step == 1:
                zx.wait_recv()
                acc = compute_slot(4, partner, acc)

        out_ref[...] = acc

        for r in cw + ccw + [zx]:
            r.wait_send()

    return pl.pallas_call(
        body,
        out_shape=jax.ShapeDtypeStruct((n_tok, h_dim), jnp.float32),
        in_specs=[pl.BlockSpec(memory_space=pltpu.VMEM)] * 5,
        out_specs=pl.BlockSpec(memory_space=pltpu.VMEM),
        scratch_shapes=[
            pltpu.VMEM((N_DEV, e_per, d, h_dim), jnp.bfloat16),
            pltpu.SemaphoreType.DMA((8,)),
            pltpu.SemaphoreType.DMA((8,)),
        ],
        compiler_params=pltpu.CompilerParams(collective_id=0),
    )(x, router_W, route_idx, expert_W, shared_W)


# device time: 58413 ns/iter; 1.0866x vs baseline; 1.0866x over previous
import jax
import jax.numpy as jnp
from jax import lax
from jax.experimental import pallas as pl
from jax.experimental.pallas import tpu as pltpu

N_DEV = 8


def kernel(x, router_W, route_idx, expert_W, shared_W):
    n_tok, d = x.shape
    e_per, _, h_dim = expert_W.shape
    n_exp = N_DEV * e_per

    def body(x_ref, rw_ref, idx_ref, ew_ref, sw_ref, out_ref,
             comm_ref, send_sems, recv_sems):
        my = lax.axis_index("i")
        left = lax.rem(my - 1 + N_DEV, N_DEV)
        right = lax.rem(my + 1, N_DEV)
        partner = lax.rem(my + 4, N_DEV)

        barrier_sem = pltpu.get_barrier_semaphore()
        for nbr in (left, right):
            pl.semaphore_signal(
                barrier_sem, inc=1,
                device_id=(nbr,), device_id_type=pl.DeviceIdType.MESH,
            )
        pl.semaphore_wait(barrier_sem, 2)

        comm_ref[0, :, :, :] = ew_ref[...].astype(jnp.bfloat16)

        half = e_per // 2
        cw = [
            pltpu.make_async_remote_copy(
                src_ref=(comm_ref.at[h - 1] if h < 4
                         else comm_ref.at[3, pl.ds(0, half)]),
                dst_ref=(comm_ref.at[h] if h < 4
                         else comm_ref.at[4, pl.ds(0, half)]),
                send_sem=send_sems.at[h - 1],
                recv_sem=recv_sems.at[h - 1],
                device_id=(right,),
                device_id_type=pl.DeviceIdType.MESH,
            )
            for h in range(1, 5)
        ]
        ccw = [
            pltpu.make_async_remote_copy(
                src_ref=(comm_ref.at[0] if h == 1
                         else comm_ref.at[4 + h - 1] if h < 4
                         else comm_ref.at[7, pl.ds(half, half)]),
                dst_ref=(comm_ref.at[4 + h] if h < 4
                         else comm_ref.at[4, pl.ds(half, half)]),
                send_sem=send_sems.at[4 + h - 1],
                recv_sem=recv_sems.at[4 + h - 1],
                device_id=(left,),
                device_id_type=pl.DeviceIdType.MESH,
            )
            for h in range(1, 5)
        ]

        cw[0].start()
        ccw[0].start()

        xf = x_ref[...]
        xbf = xf.astype(jnp.bfloat16)

        scores = jnp.dot(xf, rw_ref[...], preferred_element_type=jnp.float32)
        m = jnp.max(scores, axis=-1, keepdims=True)
        p = jnp.exp(scores - m)
        p = p / jnp.sum(p, axis=-1, keepdims=True)

        idx = idx_ref[...]
        col = lax.broadcasted_iota(jnp.int32, (n_tok, n_exp), 1)
        p_top = jnp.sum(jnp.where(col == idx, p, 0.0), axis=-1, keepdims=True)

        acc = jnp.dot(xbf, sw_ref[...].astype(jnp.bfloat16),
                      preferred_element_type=jnp.float32)

        xs_bf = (p_top * xf).astype(jnp.bfloat16)
        zero_bf = jnp.zeros_like(xs_bf)

        def compute_slot(slot, src_dev, acc):
            e0 = src_dev * e_per
            xcat = jnp.concatenate(
                [jnp.where(idx == e0 + j, xs_bf, zero_bf)
                 for j in range(e_per)], axis=1)
            w = comm_ref[slot].reshape(e_per * d, h_dim)
            return acc + jnp.dot(xcat, w, preferred_element_type=jnp.float32)

        acc = compute_slot(0, my, acc)

        for step in range(1, 4):
            cw[step - 1].wait_recv()
            cw[step].start()
            ccw[step - 1].wait_recv()
            ccw[step].start()
            acc = compute_slot(step, lax.rem(my - step + N_DEV, N_DEV), acc)
            acc = compute_slot(4 + step, lax.rem(my + step, N_DEV), acc)

        cw[3].wait_recv()
        ccw[3].wait_recv()
        acc = compute_slot(4, partner, acc)

        out_ref[...] = acc

        for r in cw + ccw:
            r.wait_send()

    return pl.pallas_call(
        body,
        out_shape=jax.ShapeDtypeStruct((n_tok, h_dim), jnp.float32),
        in_specs=[pl.BlockSpec(memory_space=pltpu.VMEM)] * 5,
        out_specs=pl.BlockSpec(memory_space=pltpu.VMEM),
        scratch_shapes=[
            pltpu.VMEM((N_DEV, e_per, d, h_dim), jnp.bfloat16),
            pltpu.SemaphoreType.DMA((8,)),
            pltpu.SemaphoreType.DMA((8,)),
        ],
        compiler_params=pltpu.CompilerParams(collective_id=0),
    )(x, router_W, route_idx, expert_W, shared_W)


# device time: 49808 ns/iter; 1.2743x vs baseline; 1.1728x over previous
import jax
import jax.numpy as jnp
from jax import lax
from jax.experimental import pallas as pl
from jax.experimental.pallas import tpu as pltpu

N_DEV = 8
PLANE = 4


def kernel(x, router_W, route_idx, expert_W, shared_W):
    n_tok, d = x.shape
    e_per, _, h_dim = expert_W.shape
    n_exp = N_DEV * e_per

    def body(x_ref, rw_ref, idx_ref, ew_ref, sw_ref, out_ref,
             comm_ref, send_sems, recv_sems):
        my = lax.axis_index("i")
        p = lax.rem(my, PLANE)
        base = my - p
        right = base + lax.rem(p + 1, PLANE)
        left = base + lax.rem(p + 3, PLANE)
        partner = lax.rem(my + 4, N_DEV)
        dist2 = base + lax.rem(p + 2, PLANE)

        barrier_sem = pltpu.get_barrier_semaphore()
        for nbr in (left, right, partner):
            pl.semaphore_signal(
                barrier_sem, inc=1,
                device_id=(nbr,), device_id_type=pl.DeviceIdType.MESH,
            )
        pl.semaphore_wait(barrier_sem, 3)

        comm_ref[0, :, :, :] = ew_ref[...].astype(jnp.bfloat16)

        half = e_per // 2

        def rdma(src_slot, dst_slot, sem, dev, lo=None):
            s = comm_ref.at[src_slot] if lo is None else \
                comm_ref.at[src_slot, pl.ds(lo, half)]
            t = comm_ref.at[dst_slot] if lo is None else \
                comm_ref.at[dst_slot, pl.ds(lo, half)]
            return pltpu.make_async_remote_copy(
                src_ref=s, dst_ref=t,
                send_sem=send_sems.at[sem], recv_sem=recv_sems.at[sem],
                device_id=(dev,), device_id_type=pl.DeviceIdType.MESH,
            )

        cwA = rdma(0, 1, 0, right)
        cwC = rdma(1, 3, 1, right)
        cwB1 = rdma(4, 2, 2, right, lo=0)
        cwB2 = rdma(4, 2, 3, right, lo=half)
        ccwD = rdma(0, 5, 4, left)
        ccwE1 = rdma(4, 6, 5, left, lo=0)
        ccwE2 = rdma(4, 6, 6, left, lo=half)
        ccwF1 = rdma(6, 7, 7, left, lo=0)
        ccwF2 = rdma(6, 7, 8, left, lo=half)
        zG = rdma(0, 4, 9, partner)

        cwA.start()
        ccwD.start()
        zG.start()

        xf = x_ref[...]
        xbf = xf.astype(jnp.bfloat16)

        scores = jnp.dot(xf, rw_ref[...], preferred_element_type=jnp.float32)
        m = jnp.max(scores, axis=-1, keepdims=True)
        p_sm = jnp.exp(scores - m)
        p_sm = p_sm / jnp.sum(p_sm, axis=-1, keepdims=True)

        idx = idx_ref[...]
        col = lax.broadcasted_iota(jnp.int32, (n_tok, n_exp), 1)
        p_top = jnp.sum(jnp.where(col == idx, p_sm, 0.0), axis=-1,
                        keepdims=True)

        acc = jnp.dot(xbf, sw_ref[...].astype(jnp.bfloat16),
                      preferred_element_type=jnp.float32)

        xs_bf = (p_top * xf).astype(jnp.bfloat16)
        zero_bf = jnp.zeros_like(xs_bf)

        def compute(slot, src_dev, acc, lo=0, n=e_per):
            e0 = src_dev * e_per + lo
            xcat = jnp.concatenate(
                [jnp.where(idx == e0 + j, xs_bf, zero_bf)
                 for j in range(n)], axis=1)
            w = comm_ref[slot, lo:lo + n].reshape(n * d, h_dim)
            return acc + jnp.dot(xcat, w, preferred_element_type=jnp.float32)

        acc = compute(0, my, acc)

        cwA.wait_recv()
        cwC.start()
        zG.wait_recv()
        cwB1.start()
        cwB2.start()
        ccwE1.start()
        ccwE2.start()
        ccwD.wait_recv()
        acc = compute(1, left, acc)
        acc = compute(5, right, acc)
        acc = compute(4, partner, acc)

        ccwE1.wait_recv()
        ccwF1.start()
        ccwE2.wait_recv()
        ccwF2.start()
        acc = compute(6, lax.rem(right + 4, N_DEV), acc)

        cwC.wait_recv()
        acc = compute(3, dist2, acc)

        dist2_partner = lax.rem(dist2 + 4, N_DEV)
        left_partner = lax.rem(left + 4, N_DEV)
        cwB1.wait_recv()
        acc = compute(2, left_partner, acc, lo=0, n=half)
        ccwF1.wait_recv()
        acc = compute(7, dist2_partner, acc, lo=0, n=half)
        cwB2.wait_recv()
        acc = compute(2, left_partner, acc, lo=half, n=half)
        ccwF2.wait_recv()
        acc = compute(7, dist2_partner, acc, lo=half, n=half)

        out_ref[...] = acc

        for r in (cwA, cwC, cwB1, cwB2, ccwD, ccwE1, ccwE2,
                  ccwF1, ccwF2, zG):
            r.wait_send()

    return pl.pallas_call(
        body,
        out_shape=jax.ShapeDtypeStruct((n_tok, h_dim), jnp.float32),
        in_specs=[pl.BlockSpec(memory_space=pltpu.VMEM)] * 5,
        out_specs=pl.BlockSpec(memory_space=pltpu.VMEM),
        scratch_shapes=[
            pltpu.VMEM((N_DEV, e_per, d, h_dim), jnp.bfloat16),
            pltpu.SemaphoreType.DMA((10,)),
            pltpu.SemaphoreType.DMA((10,)),
        ],
        compiler_params=pltpu.CompilerParams(collective_id=0),
    )(x, router_W, route_idx, expert_W, shared_W)


# device time: 45506 ns/iter; 1.3948x vs baseline; 1.0945x over previous
import jax
import jax.numpy as jnp
from jax import lax
from jax.experimental import pallas as pl
from jax.experimental.pallas import tpu as pltpu

N_DEV = 8
PLANE = 4


def kernel(x, router_W, route_idx, expert_W, shared_W):
    n_tok, d = x.shape
    e_per, _, h_dim = expert_W.shape
    n_exp = N_DEV * e_per

    def body(x_ref, rw_ref, idx_ref, ew_ref, sw_ref, out_ref,
             comm_ref, send_sems, recv_sems):
        my = lax.axis_index("i")
        p = lax.rem(my, PLANE)
        base = my - p
        right = base + lax.rem(p + 1, PLANE)
        left = base + lax.rem(p + 3, PLANE)
        partner = lax.rem(my + 4, N_DEV)
        dist2 = base + lax.rem(p + 2, PLANE)

        barrier_sem = pltpu.get_barrier_semaphore()
        for nbr in (left, right, partner):
            pl.semaphore_signal(
                barrier_sem, inc=1,
                device_id=(nbr,), device_id_type=pl.DeviceIdType.MESH,
            )
        pl.semaphore_wait(barrier_sem, 3)

        comm_ref[0, :, :, :] = ew_ref[...].astype(jnp.bfloat16)

        half = e_per // 2

        def rdma(src_slot, dst_slot, sem, dev, lo=None):
            s = comm_ref.at[src_slot] if lo is None else \
                comm_ref.at[src_slot, pl.ds(lo, half)]
            t = comm_ref.at[dst_slot] if lo is None else \
                comm_ref.at[dst_slot, pl.ds(lo, half)]
            return pltpu.make_async_remote_copy(
                src_ref=s, dst_ref=t,
                send_sem=send_sems.at[sem], recv_sem=recv_sems.at[sem],
                device_id=(dev,), device_id_type=pl.DeviceIdType.MESH,
            )

        cwA = rdma(0, 1, 0, right)
        cwC = rdma(1, 3, 1, right)
        cwB1 = rdma(4, 2, 2, right, lo=0)
        ccwD = rdma(0, 5, 3, left)
        ccwE1 = rdma(4, 6, 4, left, lo=0)
        ccwF1 = rdma(6, 7, 5, left, lo=0)
        ccwF2 = rdma(6, 7, 6, left, lo=half)
        zG = rdma(0, 4, 7, partner)
        zH = rdma(1, 2, 8, partner, lo=half)
        zI = rdma(5, 6, 9, partner, lo=half)

        cwA.start()
        ccwD.start()
        zG.start()

        xf = x_ref[...]
        xbf = xf.astype(jnp.bfloat16)

        scores = jnp.dot(xf, rw_ref[...], preferred_element_type=jnp.float32)
        m = jnp.max(scores, axis=-1, keepdims=True)
        p_sm = jnp.exp(scores - m)
        p_sm = p_sm / jnp.sum(p_sm, axis=-1, keepdims=True)

        idx = idx_ref[...]
        col = lax.broadcasted_iota(jnp.int32, (n_tok, n_exp), 1)
        p_top = jnp.sum(jnp.where(col == idx, p_sm, 0.0), axis=-1,
                        keepdims=True)

        acc = jnp.dot(xbf, sw_ref[...].astype(jnp.bfloat16),
                      preferred_element_type=jnp.float32)

        xs_bf = (p_top * xf).astype(jnp.bfloat16)
        zero_bf = jnp.zeros_like(xs_bf)

        def compute(slot, src_dev, acc, lo=0, n=e_per):
            e0 = src_dev * e_per + lo
            xcat = jnp.concatenate(
                [jnp.where(idx == e0 + j, xs_bf, zero_bf)
                 for j in range(n)], axis=1)
            w = comm_ref[slot, lo:lo + n].reshape(n * d, h_dim)
            return acc + jnp.dot(xcat, w, preferred_element_type=jnp.float32)

        acc = compute(0, my, acc)

        dist2_partner = lax.rem(dist2 + 4, N_DEV)
        left_partner = lax.rem(left + 4, N_DEV)
        right_partner = lax.rem(right + 4, N_DEV)

        cwA.wait_recv()
        cwC.start()
        zH.start()
        zG.wait_recv()
        cwB1.start()
        ccwE1.start()
        ccwD.wait_recv()
        zI.start()
        acc = compute(1, left, acc)
        acc = compute(5, right, acc)
        acc = compute(4, partner, acc)

        zH.wait_recv()
        acc = compute(2, left_partner, acc, lo=half, n=half)
        ccwE1.wait_recv()
        ccwF1.start()
        acc = compute(6, right_partner, acc, lo=0, n=half)
        zI.wait_recv()
        ccwF2.start()
        acc = compute(6, right_partner, acc, lo=half, n=half)
        cwC.wait_recv()
        acc = compute(3, dist2, acc)
        ccwF1.wait_recv()
        acc = compute(7, dist2_partner, acc, lo=0, n=half)
        cwB1.wait_recv()
        acc = compute(2, left_partner, acc, lo=0, n=half)
        ccwF2.wait_recv()
        acc = compute(7, dist2_partner, acc, lo=half, n=half)

        out_ref[...] = acc

        for r in (cwA, cwC, cwB1, ccwD, ccwE1, ccwF1, ccwF2,
                  zG, zH, zI):
            r.wait_send()

    return pl.pallas_call(
        body,
        out_shape=jax.ShapeDtypeStruct((n_tok, h_dim), jnp.float32),
        in_specs=[pl.BlockSpec(memory_space=pltpu.VMEM)] * 5,
        out_specs=pl.BlockSpec(memory_space=pltpu.VMEM),
        scratch_shapes=[
            pltpu.VMEM((N_DEV, e_per, d, h_dim), jnp.bfloat16),
            pltpu.SemaphoreType.DMA((10,)),
            pltpu.SemaphoreType.DMA((10,)),
        ],
        compiler_params=pltpu.CompilerParams(collective_id=0),
    )(x, router_W, route_idx, expert_W, shared_W)
